# Initial kernel scaffold; baseline (speedup 1.0000x reference)
import jax
import jax.numpy as jnp
from jax.experimental import pallas as pl


def kernel(h, edge_index, M, W_h, W_s):
    raise NotImplementedError("write your pallas kernel here")


# speedup vs baseline: 13.9612x; 13.9612x over previous
import functools

import jax
import jax.numpy as jnp
from jax import lax
from jax.experimental import pallas as pl
from jax.experimental.pallas import tpu as pltpu
from jax.experimental.pallas import tpu_sc as plsc

N = 100000
E = 1600000
D = 128
K = 32
TAU = 1.0
ALPHA = 1.0

BN = 2000
NB = N // BN
SINK_PASSES = 4
R = 1.0 / K
C = 1.0 / N

NW = 32
EPW = E // NW
EB = 1000
NCHUNK = EPW // EB
U = 8


def _q_body(h_ref, wh_ref, m_ref, q_ref, rs_ref):
    h = h_ref[...]
    wh = wh_ref[...]
    m = m_ref[...]
    hw = jnp.dot(h, wh.T, preferred_element_type=jnp.float32)
    hn = jnp.sqrt(jnp.sum(hw * hw, axis=1, keepdims=True))
    hw = hw / jnp.maximum(hn, 1e-8)
    mn = jnp.sqrt(jnp.sum(m * m, axis=1, keepdims=True))
    mm = m / jnp.maximum(mn, 1e-8)
    c = jnp.dot(hw, mm.T, preferred_element_type=jnp.float32) / TAU
    cmax = jnp.max(c, axis=1, keepdims=True)
    e = jnp.exp(c - cmax)
    q = e / jnp.sum(e, axis=1, keepdims=True)
    q_ref[...] = q
    part = jnp.sum(jnp.exp(20.0 * q), axis=0, keepdims=True)

    @pl.when(pl.program_id(0) == 0)
    def _():
        rs_ref[...] = part

    @pl.when(pl.program_id(0) != 0)
    def _():
        rs_ref[...] += part


def _compute_q(h, W_h, M):
    return pl.pallas_call(
        _q_body,
        grid=(NB,),
        in_specs=[
            pl.BlockSpec((BN, D), lambda i: (i, 0)),
            pl.BlockSpec((D, D), lambda i: (0, 0)),
            pl.BlockSpec((K, D), lambda i: (0, 0)),
        ],
        out_specs=[
            pl.BlockSpec((BN, K), lambda i: (i, 0)),
            pl.BlockSpec((1, K), lambda i: (0, 0)),
        ],
        out_shape=[
            jax.ShapeDtypeStruct((N, K), jnp.float32),
            jax.ShapeDtypeStruct((1, K), jnp.float32),
        ],
    )(h, W_h, M)


def _sink_body(rs_ref, q_ref, uout_ref, u_s, t_s):
    p = pl.program_id(0)
    i = pl.program_id(1)

    @pl.when(i == 0)
    def _():
        @pl.when(p == 0)
        def _():
            u_s[...] = R / rs_ref[...]

        @pl.when(p != 0)
        def _():
            u_s[...] = R / t_s[...]

        t_s[...] = jnp.zeros_like(t_s)

    a = jnp.exp(20.0 * q_ref[...])
    w = jnp.sum(a * u_s[...], axis=1, keepdims=True)
    v = C / w
    t_s[...] += jnp.sum(a * v, axis=0, keepdims=True)

    @pl.when(jnp.logical_and(p == SINK_PASSES - 1, i == NB - 1))
    def _():
        uout_ref[...] = R / t_s[...]


def _sinkhorn_u(rowsum, q):
    return pl.pallas_call(
        _sink_body,
        grid=(SINK_PASSES, NB),
        in_specs=[
            pl.BlockSpec((1, K), lambda p, i: (0, 0)),
            pl.BlockSpec((BN, K), lambda p, i: (i, 0)),
        ],
        out_specs=pl.BlockSpec((1, K), lambda p, i: (0, 0)),
        out_shape=jax.ShapeDtypeStruct((1, K), jnp.float32),
        scratch_shapes=[
            pltpu.VMEM((1, K), jnp.float32),
            pltpu.VMEM((1, K), jnp.float32),
        ],
    )(rowsum, q)


def _pool_body(q_ref, h_ref, u_ref, s_ref, cnt_ref, nl_ref):
    i = pl.program_id(0)
    q = q_ref[...]
    h = h_ref[...]
    u = u_ref[...]
    score = 20.0 * q + jnp.log(u)
    iota = lax.broadcasted_iota(jnp.int32, (1, K), 1)
    m = jnp.max(score, axis=1, keepdims=True)
    is_max = score >= m
    first = jnp.min(jnp.where(is_max, iota, K), axis=1, keepdims=True)
    onehot = (iota == first).astype(jnp.float32)
    s_part = lax.dot_general(onehot, h, (((0,), (0,)), ((), ())),
                             preferred_element_type=jnp.float32)
    cnt_part = jnp.sum(onehot, axis=0, keepdims=True)
    qsel = jnp.sum(onehot * q, axis=1, keepdims=True)
    nl_part = jnp.sum(jnp.log(qsel + 1e-12)).reshape(1, 1)

    @pl.when(i == 0)
    def _():
        s_ref[...] = s_part
        cnt_ref[...] = cnt_part
        nl_ref[...] = nl_part

    @pl.when(i != 0)
    def _():
        s_ref[...] += s_part
        cnt_ref[...] += cnt_part
        nl_ref[...] += nl_part


def _pool(q, h, u):
    return pl.pallas_call(
        _pool_body,
        grid=(NB,),
        in_specs=[
            pl.BlockSpec((BN, K), lambda i: (i, 0)),
            pl.BlockSpec((BN, D), lambda i: (i, 0)),
            pl.BlockSpec((1, K), lambda i: (0, 0)),
        ],
        out_specs=[
            pl.BlockSpec((K, D), lambda i: (0, 0)),
            pl.BlockSpec((1, K), lambda i: (0, 0)),
            pl.BlockSpec((1, 1), lambda i: (0, 0)),
        ],
        out_shape=[
            jax.ShapeDtypeStruct((K, D), jnp.float32),
            jax.ShapeDtypeStruct((1, K), jnp.float32),
            jax.ShapeDtypeStruct((1, 1), jnp.float32),
        ],
    )(q, h, u)


def _edge_body(q_hbm, src_hbm, dst_hbm, out_hbm, sidx, didx, srows, drows, accv, sem):
    wid = lax.axis_index("s") * 2 + lax.axis_index("c")
    base = wid * EPW

    def chunk(ci, accs):
        off = base + ci * EB
        pltpu.sync_copy(src_hbm.at[pl.ds(off, EB)], sidx)
        pltpu.sync_copy(dst_hbm.at[pl.ds(off, EB)], didx)
        cp1 = pltpu.async_copy(q_hbm.at[sidx], srows, sem)
        cp2 = pltpu.async_copy(q_hbm.at[didx], drows, sem)
        cp1.wait()
        cp2.wait()

        def inner(j, accs):
            accs = list(accs)
            for e in range(U):
                i = j * U + e
                p0 = srows[i, 0:16] * drows[i, 0:16]
                p1 = srows[i, 16:32] * drows[i, 16:32]
                accs[e] = accs[e] + p0 + p1
            return tuple(accs)

        return lax.fori_loop(0, EB // U, inner, accs)

    zero = jnp.zeros((16,), jnp.float32)
    accs = lax.fori_loop(0, NCHUNK, chunk, (zero,) * U)
    total = accs[0]
    for e in range(1, U):
        total = total + accs[e]
    accv[...] = total
    pltpu.sync_copy(accv, out_hbm.at[wid])


def _edge_partials(q, edge_index):
    mesh = plsc.VectorSubcoreMesh(core_axis_name="c", subcore_axis_name="s")
    f = functools.partial(
        pl.kernel,
        mesh=mesh,
        compiler_params=pltpu.CompilerParams(use_tc_tiling_on_sc=False),
        out_type=jax.ShapeDtypeStruct((NW, 16), jnp.float32),
        scratch_types=[
            pltpu.VMEM((EB,), jnp.int32),
            pltpu.VMEM((EB,), jnp.int32),
            pltpu.VMEM((EB, K), jnp.float32),
            pltpu.VMEM((EB, K), jnp.float32),
            pltpu.VMEM((16,), jnp.float32),
            pltpu.SemaphoreType.DMA,
        ],
    )(_edge_body)
    return f(q, edge_index[0], edge_index[1])


def _final_body(s_ref, cnt_ref, nl_ref, ep_ref, ws_ref, m_ref, out_ref):
    cnt = jnp.maximum(cnt_ref[...], 1.0)
    iota_r = lax.broadcasted_iota(jnp.int32, (K, K), 0)
    iota_c = lax.broadcasted_iota(jnp.int32, (K, K), 1)
    eye = (iota_r == iota_c).astype(jnp.float32)
    dinv = eye * (1.0 / cnt)
    s = jnp.dot(dinv, s_ref[...], preferred_element_type=jnp.float32)
    sw = jnp.dot(s, ws_ref[...].T, preferred_element_type=jnp.float32)
    sn = jnp.sqrt(jnp.sum(sw * sw, axis=1, keepdims=True))
    sw = sw / jnp.maximum(sn, 1e-8)
    m = m_ref[...]
    mn = jnp.sqrt(jnp.sum(m * m, axis=1, keepdims=True))
    mm = m / jnp.maximum(mn, 1e-8)
    c = jnp.dot(sw, mm.T, preferred_element_type=jnp.float32) / TAU
    cmax = jnp.max(c, axis=1, keepdims=True)
    e = jnp.exp(c - cmax)
    p = e / jnp.sum(e, axis=1, keepdims=True)
    diag = jnp.sum(p * eye, axis=1, keepdims=True)
    loss_mot_sub = -jnp.sum(jnp.log(diag + 1e-12), keepdims=True).reshape(1, 1) / K
    loss_node_mot = -nl_ref[...] / N
    loss_reg = -jnp.sum(ep_ref[...], keepdims=True).reshape(1, 1) / E
    out_ref[...] = ALPHA * (loss_mot_sub + loss_node_mot + loss_reg)


def _final(s_raw, cnt, nl, edge_parts, W_s, M):
    return pl.pallas_call(
        _final_body,
        out_shape=jax.ShapeDtypeStruct((1, 1), jnp.float32),
    )(s_raw, cnt, nl, edge_parts, W_s, M)


def kernel(h, edge_index, M, W_h, W_s):
    q, rowsum = _compute_q(h, W_h, M)
    edge_parts = _edge_partials(q, edge_index)
    u = _sinkhorn_u(rowsum, q)
    s_raw, cnt, nl = _pool(q, h, u)
    out = _final(s_raw, cnt, nl, edge_parts, W_s, M)
    return out.reshape(())
